# baseline (speedup 1.0000x reference)
import jax
import jax.numpy as jnp
from jax import lax
from jax.experimental import pallas as pl
from jax.experimental.pallas import tpu as pltpu
from jax.experimental.pallas import tpu_sc as plsc

NUM_ATOMS = 100000
NUM_ELEMENTS = 119
EMBED_DIM = 256
BLK = 80
NB = NUM_ATOMS // BLK
NW = 32
BPW = (NB + NW - 1) // NW


def _body(idx_hbm, table_hbm, out_hbm,
          idx_v, rows0, rows1, gsem, wsem0, wsem1):
    c = lax.axis_index("c")
    s = lax.axis_index("s")
    w = s * 2 + c
    start = w * BPW
    nb_w = jnp.minimum(BPW, NB - start)

    pltpu.sync_copy(idx_hbm.at[pl.ds(start, BPW)], idx_v)

    bufs = (rows0, rows1)
    wsems = (wsem0, wsem1)

    def pair(j, carry):
        for p in range(2):
            b = 2 * j + p

            @pl.when(b < nb_w)
            def _():
                @pl.when(j >= 1)
                def _():
                    pltpu.make_async_copy(
                        bufs[p],
                        out_hbm.at[pl.ds(0, BLK)],
                        wsems[p]).wait()

                pltpu.async_copy(table_hbm.at[idx_v.at[b]], bufs[p],
                                 gsem).wait()
                pltpu.async_copy(bufs[p],
                                 out_hbm.at[pl.ds((start + b) * BLK, BLK)],
                                 wsems[p])

        return carry

    lax.fori_loop(0, (BPW + 1) // 2, pair, 0)

    for p in range(2):
        pltpu.make_async_copy(bufs[p], out_hbm.at[pl.ds(0, BLK)],
                              wsems[p]).wait()


def kernel(atomic_numbers, embedding):
    mesh = plsc.VectorSubcoreMesh(core_axis_name="c", subcore_axis_name="s")
    k = pl.kernel(
        _body,
        mesh=mesh,
        out_type=jax.ShapeDtypeStruct((NUM_ATOMS, EMBED_DIM), jnp.float32),
        scratch_types=[
            pltpu.VMEM((BPW, BLK), jnp.int32),
            pltpu.VMEM((BLK, EMBED_DIM), jnp.float32),
            pltpu.VMEM((BLK, EMBED_DIM), jnp.float32),
            pltpu.SemaphoreType.DMA,
            pltpu.SemaphoreType.DMA,
            pltpu.SemaphoreType.DMA,
        ],
    )
    idx2d = atomic_numbers.astype(jnp.int32).reshape(NB, BLK)
    idx2d = jnp.pad(idx2d, ((0, NW * BPW - NB), (0, 0)))
    return k(idx2d, embedding)
